# baseline (speedup 1.0000x reference)
import jax
import jax.numpy as jnp
import numpy as np
from jax import lax
from jax.experimental import pallas as pl
from jax.experimental.pallas import tpu as pltpu
from jax.experimental.pallas import tpu_sc as plsc

NC = 2
NS = 16
L = 16
NW = NC * NS

H = W = 512
PIX_PER_SAMPLE = H * W
BATCH = 8
TOTAL_PIX = BATCH * PIX_PER_SAMPLE
PIX_PER_WORKER = TOTAL_PIX // NW
CH = 2048
NCHUNK = PIX_PER_WORKER // CH
NIDX = 128
NSTREAM = CH // NIDX
NVREG = CH // L

ZERO_W = 3 * TOTAL_PIX
MAGIC = np.float32(12582912.0)


def _rne(x):
    return (x + MAGIC) - MAGIC


def _body(tex_hbm, lut_hbm, iuv_hbm, out_hbm,
          iuv0, iuv1, iuv2, la, lb, arows, brows, t0, t1, t2,
          oc0, oc1, oc2, sem):
    wid = lax.axis_index("s") * NC + lax.axis_index("c")
    n = wid // 4
    q = wid % 4
    n_base_f = (n * PIX_PER_SAMPLE).astype(jnp.float32)
    plane = n * (3 * PIX_PER_SAMPLE)
    iota = lax.iota(jnp.int32, L)

    def gather_stage(tbl, idx_ref, dst_ref):
        def fire(j, _):
            pltpu.async_copy(tbl.at[idx_ref.at[pl.ds(j * NIDX, NIDX)]],
                             dst_ref.at[pl.ds(j * NIDX, NIDX)], sem)
            return 0
        lax.fori_loop(0, NSTREAM, fire, 0)

        def drain(j, _):
            pltpu.make_async_copy(
                tbl.at[idx_ref.at[pl.ds(j * NIDX, NIDX)]],
                dst_ref.at[pl.ds(j * NIDX, NIDX)], sem).wait()
            return 0
        lax.fori_loop(0, NSTREAM, drain, 0)

    def chunk_body(c, _):
        base = q * PIX_PER_WORKER + c * CH

        pltpu.sync_copy(iuv_hbm.at[pl.ds(plane + base, CH)], iuv0)
        pltpu.sync_copy(iuv_hbm.at[pl.ds(plane + PIX_PER_SAMPLE + base, CH)],
                        iuv1)
        pltpu.sync_copy(iuv_hbm.at[pl.ds(plane + 2 * PIX_PER_SAMPLE + base,
                                         CH)], iuv2)

        def pass_a(i, _):
            s = pl.ds(i * L, L)
            p0 = iuv0[s]
            p1 = iuv1[s]
            p2 = iuv2[s]
            part = jnp.where(p0 > 0, p0 - 1, 0)
            lw = ((part << 16) | (p2 << 8) | p1) << 1
            la[s] = lw
            lb[s] = lw | 1
            return 0
        lax.fori_loop(0, NVREG, pass_a, 0)

        gather_stage(lut_hbm, la, arows)
        gather_stage(lut_hbm, lb, brows)

        def pass_b(i, _):
            s = pl.ds(i * L, L)
            a = arows[s]
            b = brows[s]
            u_i = _rne(a * np.float32(511.0))
            v_i = _rne((np.float32(1.0) - b) * np.float32(511.0))
            tf = v_i * np.float32(512.0) + u_i + n_base_f
            ti = tf.astype(jnp.int32)
            tw = (ti << 1) + ti
            p0 = iuv0[s]
            w0 = jnp.where(p0 > 0, tw, ZERO_W)
            t0[s] = w0
            t1[s] = w0 + 1
            t2[s] = w0 + 2
            return 0
        lax.fori_loop(0, NVREG, pass_b, 0)

        gather_stage(tex_hbm, t0, oc0)
        gather_stage(tex_hbm, t1, oc1)
        gather_stage(tex_hbm, t2, oc2)

        pltpu.sync_copy(oc0, out_hbm.at[pl.ds(plane + base, CH)])
        pltpu.sync_copy(oc1, out_hbm.at[pl.ds(plane + PIX_PER_SAMPLE + base,
                                              CH)])
        pltpu.sync_copy(oc2, out_hbm.at[pl.ds(plane + 2 * PIX_PER_SAMPLE
                                              + base, CH)])
        return 0

    lax.fori_loop(0, NCHUNK, chunk_body, 0)


@jax.jit
def kernel(img_or_tex, iuv_img, lut):
    tex_tbl = jnp.pad(
        img_or_tex.transpose(0, 2, 3, 1).reshape(3 * TOTAL_PIX), ((0, 3),))
    lut_tbl = lut.reshape(2 * 24 * 256 * 256)
    iuv = iuv_img.reshape(BATCH * 3 * PIX_PER_SAMPLE)

    mesh = plsc.VectorSubcoreMesh(
        core_axis_name="c", subcore_axis_name="s",
        num_cores=NC, num_subcores=NS)
    run = pl.kernel(
        _body,
        out_type=jax.ShapeDtypeStruct((BATCH * 3 * PIX_PER_SAMPLE,),
                                      jnp.float32),
        mesh=mesh,
        compiler_params=pltpu.CompilerParams(
            needs_layout_passes=False, use_tc_tiling_on_sc=False),
        scratch_types=[
            pltpu.VMEM((CH,), jnp.int32),
            pltpu.VMEM((CH,), jnp.int32),
            pltpu.VMEM((CH,), jnp.int32),
            pltpu.VMEM((CH,), jnp.int32),
            pltpu.VMEM((CH,), jnp.int32),
            pltpu.VMEM((CH,), jnp.float32),
            pltpu.VMEM((CH,), jnp.float32),
            pltpu.VMEM((CH,), jnp.int32),
            pltpu.VMEM((CH,), jnp.int32),
            pltpu.VMEM((CH,), jnp.int32),
            pltpu.VMEM((CH,), jnp.float32),
            pltpu.VMEM((CH,), jnp.float32),
            pltpu.VMEM((CH,), jnp.float32),
            pltpu.SemaphoreType.DMA,
        ],
    )
    out = run(tex_tbl, lut_tbl, iuv)
    return out.reshape(BATCH, 3, H, W)


# speedup vs baseline: 2.1585x; 2.1585x over previous
import jax
import jax.numpy as jnp
import numpy as np
from jax import lax
from jax.experimental import pallas as pl
from jax.experimental.pallas import tpu as pltpu
from jax.experimental.pallas import tpu_sc as plsc

NC = 2
NS = 16
L = 16
NW = NC * NS

H = W = 512
PIX_PER_SAMPLE = H * W
BATCH = 8
TOTAL_PIX = BATCH * PIX_PER_SAMPLE
PIX_PER_WORKER = TOTAL_PIX // NW
CH = 4096
NCHUNK = PIX_PER_WORKER // CH
NIDX = 128
NSTREAM = CH // NIDX
NVREG = CH // L

MAGIC = np.float32(12582912.0)


def _rne(x):
    return (x + MAGIC) - MAGIC


def _body(tex_hbm, lut_hbm, iuv_hbm, out_hbm,
          iuv0, iuv1, iuv2, la, lb, arows, brows, t0, t1, t2, mf,
          oc0, oc1, oc2, sem):
    wid = lax.axis_index("s") * NC + lax.axis_index("c")
    n = wid // 4
    q = wid % 4
    n_base_f = (n * 3 * PIX_PER_SAMPLE).astype(jnp.float32)
    plane = n * (3 * PIX_PER_SAMPLE)

    def fire(tbl, idx_ref, dst_ref, j):
        pltpu.async_copy(tbl.at[idx_ref.at[pl.ds(j * NIDX, NIDX)]],
                         dst_ref.at[pl.ds(j * NIDX, NIDX)], sem)

    def drain(tbl, idx_ref, dst_ref, j):
        pltpu.make_async_copy(
            tbl.at[idx_ref.at[pl.ds(j * NIDX, NIDX)]],
            dst_ref.at[pl.ds(j * NIDX, NIDX)], sem).wait()

    def chunk_body(c, _):
        base = q * PIX_PER_WORKER + c * CH

        pltpu.sync_copy(iuv_hbm.at[pl.ds(plane + base, CH)], iuv0)
        pltpu.sync_copy(iuv_hbm.at[pl.ds(plane + PIX_PER_SAMPLE + base, CH)],
                        iuv1)
        pltpu.sync_copy(iuv_hbm.at[pl.ds(plane + 2 * PIX_PER_SAMPLE + base,
                                         CH)], iuv2)

        def pass_a(i, _):
            s = pl.ds(i * L, L)
            p0 = iuv0[s]
            p1 = iuv1[s]
            p2 = iuv2[s]
            part = jnp.where(p0 > 0, p0 - 1, 0)
            lw = ((part << 16) | (p2 << 8) | p1) << 1
            la[s] = lw
            lb[s] = lw | 1
            return 0
        lax.fori_loop(0, NVREG, pass_a, 0)

        def fire_lut(j, _):
            fire(lut_hbm, la, arows, j)
            fire(lut_hbm, lb, brows, j)
            return 0
        lax.fori_loop(0, NSTREAM, fire_lut, 0)

        def drain_lut(j, _):
            drain(lut_hbm, la, arows, j)
            drain(lut_hbm, lb, brows, j)
            return 0
        lax.fori_loop(0, NSTREAM, drain_lut, 0)

        def pass_b(i, _):
            s = pl.ds(i * L, L)
            a = arows[s]
            b = brows[s]
            u_i = _rne(a * np.float32(511.0))
            v_i = _rne((np.float32(1.0) - b) * np.float32(511.0))
            tf = v_i * np.float32(512.0) + u_i + n_base_f
            ti = tf.astype(jnp.int32)
            t0[s] = ti
            t1[s] = ti + PIX_PER_SAMPLE
            t2[s] = ti + 2 * PIX_PER_SAMPLE
            p0 = iuv0[s]
            mf[s] = jnp.where(p0 > 0, np.float32(1.0), np.float32(0.0))
            return 0
        lax.fori_loop(0, NVREG, pass_b, 0)

        def fire_tex(j, _):
            fire(tex_hbm, t0, oc0, j)
            fire(tex_hbm, t1, oc1, j)
            fire(tex_hbm, t2, oc2, j)
            return 0
        lax.fori_loop(0, NSTREAM, fire_tex, 0)

        def drain_tex(j, _):
            drain(tex_hbm, t0, oc0, j)
            drain(tex_hbm, t1, oc1, j)
            drain(tex_hbm, t2, oc2, j)
            return 0
        lax.fori_loop(0, NSTREAM, drain_tex, 0)

        def pass_c(i, _):
            s = pl.ds(i * L, L)
            m = mf[s]
            oc0[s] = oc0[s] * m
            oc1[s] = oc1[s] * m
            oc2[s] = oc2[s] * m
            return 0
        lax.fori_loop(0, NVREG, pass_c, 0)

        pltpu.sync_copy(oc0, out_hbm.at[pl.ds(plane + base, CH)])
        pltpu.sync_copy(oc1, out_hbm.at[pl.ds(plane + PIX_PER_SAMPLE + base,
                                              CH)])
        pltpu.sync_copy(oc2, out_hbm.at[pl.ds(plane + 2 * PIX_PER_SAMPLE
                                              + base, CH)])
        return 0

    lax.fori_loop(0, NCHUNK, chunk_body, 0)


@jax.jit
def kernel(img_or_tex, iuv_img, lut):
    tex = img_or_tex.reshape(BATCH * 3 * PIX_PER_SAMPLE)
    lut_tbl = lut.reshape(2 * 24 * 256 * 256)
    iuv = iuv_img.reshape(BATCH * 3 * PIX_PER_SAMPLE)

    mesh = plsc.VectorSubcoreMesh(
        core_axis_name="c", subcore_axis_name="s",
        num_cores=NC, num_subcores=NS)
    run = pl.kernel(
        _body,
        out_type=jax.ShapeDtypeStruct((BATCH * 3 * PIX_PER_SAMPLE,),
                                      jnp.float32),
        mesh=mesh,
        compiler_params=pltpu.CompilerParams(
            needs_layout_passes=False, use_tc_tiling_on_sc=False),
        scratch_types=[
            pltpu.VMEM((CH,), jnp.int32),
            pltpu.VMEM((CH,), jnp.int32),
            pltpu.VMEM((CH,), jnp.int32),
            pltpu.VMEM((CH,), jnp.int32),
            pltpu.VMEM((CH,), jnp.int32),
            pltpu.VMEM((CH,), jnp.float32),
            pltpu.VMEM((CH,), jnp.float32),
            pltpu.VMEM((CH,), jnp.int32),
            pltpu.VMEM((CH,), jnp.int32),
            pltpu.VMEM((CH,), jnp.int32),
            pltpu.VMEM((CH,), jnp.float32),
            pltpu.VMEM((CH,), jnp.float32),
            pltpu.VMEM((CH,), jnp.float32),
            pltpu.VMEM((CH,), jnp.float32),
            pltpu.SemaphoreType.DMA,
        ],
    )
    out = run(tex, lut_tbl, iuv)
    return out.reshape(BATCH, 3, H, W)


# speedup vs baseline: 8.4848x; 3.9310x over previous
import jax
import jax.numpy as jnp
import numpy as np
from jax import lax
from jax.experimental import pallas as pl
from jax.experimental.pallas import tpu as pltpu
from jax.experimental.pallas import tpu_sc as plsc

NC = 2
NS = 16
L = 16
NW = NC * NS

H = W = 512
PIX_PER_SAMPLE = H * W
BATCH = 8
TOTAL_PIX = BATCH * PIX_PER_SAMPLE
PIX_PER_WORKER = TOTAL_PIX // NW
CH = 4096
NCHUNK = PIX_PER_WORKER // CH
NIDX = 128
NSTREAM = CH // NIDX
NVREG = CH // L

MAGIC = np.float32(12582912.0)


def _rne(x):
    return (x + MAGIC) - MAGIC


def _body(tex_hbm, luta_hbm, lutb_hbm, iuv_hbm, out_hbm,
          iuv0, iuv1, iuv2, la, arows, brows, t0, t1, t2, mf,
          oc0, oc1, oc2, sem):
    wid = lax.axis_index("s") * NC + lax.axis_index("c")
    n = wid // 4
    q = wid % 4
    n_base_f = (n * 3 * PIX_PER_SAMPLE).astype(jnp.float32)
    plane = n * (3 * PIX_PER_SAMPLE)

    def fire(tbl, idx_ref, dst_ref, j):
        pltpu.async_copy(tbl.at[idx_ref.at[pl.ds(j * NIDX, NIDX)]],
                         dst_ref.at[pl.ds(j * NIDX, NIDX)], sem)

    def drain(tbl, idx_ref, dst_ref, j):
        pltpu.make_async_copy(
            tbl.at[idx_ref.at[pl.ds(j * NIDX, NIDX)]],
            dst_ref.at[pl.ds(j * NIDX, NIDX)], sem).wait()

    def chunk_body(c, _):
        base = q * PIX_PER_WORKER + c * CH

        pltpu.sync_copy(iuv_hbm.at[pl.ds(plane + base, CH)], iuv0)
        pltpu.sync_copy(iuv_hbm.at[pl.ds(plane + PIX_PER_SAMPLE + base, CH)],
                        iuv1)
        pltpu.sync_copy(iuv_hbm.at[pl.ds(plane + 2 * PIX_PER_SAMPLE + base,
                                         CH)], iuv2)

        def pass_a(i, _):
            s = pl.ds(i * L, L)
            p0 = iuv0[s]
            p1 = iuv1[s]
            p2 = iuv2[s]
            part = jnp.where(p0 > 0, p0 - 1, 0)
            la[s] = (part << 16) | (p2 << 8) | p1
            return 0
        lax.fori_loop(0, NVREG, pass_a, 0)

        def fire_lut(j, _):
            fire(luta_hbm, la, arows, j)
            fire(lutb_hbm, la, brows, j)
            return 0
        lax.fori_loop(0, NSTREAM, fire_lut, 0)

        def drain_lut(j, _):
            drain(luta_hbm, la, arows, j)
            drain(lutb_hbm, la, brows, j)
            return 0
        lax.fori_loop(0, NSTREAM, drain_lut, 0)

        def pass_b(i, _):
            s = pl.ds(i * L, L)
            a = arows[s]
            b = brows[s]
            u_i = _rne(a * np.float32(511.0))
            v_i = _rne((np.float32(1.0) - b) * np.float32(511.0))
            tf = v_i * np.float32(512.0) + u_i + n_base_f
            ti = tf.astype(jnp.int32)
            t0[s] = ti
            t1[s] = ti + PIX_PER_SAMPLE
            t2[s] = ti + 2 * PIX_PER_SAMPLE
            p0 = iuv0[s]
            mf[s] = jnp.where(p0 > 0, np.float32(1.0), np.float32(0.0))
            return 0
        lax.fori_loop(0, NVREG, pass_b, 0)

        def fire_tex(j, _):
            fire(tex_hbm, t0, oc0, j)
            fire(tex_hbm, t1, oc1, j)
            fire(tex_hbm, t2, oc2, j)
            return 0
        lax.fori_loop(0, NSTREAM, fire_tex, 0)

        def drain_tex(j, _):
            drain(tex_hbm, t0, oc0, j)
            drain(tex_hbm, t1, oc1, j)
            drain(tex_hbm, t2, oc2, j)
            return 0
        lax.fori_loop(0, NSTREAM, drain_tex, 0)

        def pass_c(i, _):
            s = pl.ds(i * L, L)
            m = mf[s]
            oc0[s] = oc0[s] * m
            oc1[s] = oc1[s] * m
            oc2[s] = oc2[s] * m
            return 0
        lax.fori_loop(0, NVREG, pass_c, 0)

        pltpu.sync_copy(oc0, out_hbm.at[pl.ds(plane + base, CH)])
        pltpu.sync_copy(oc1, out_hbm.at[pl.ds(plane + PIX_PER_SAMPLE + base,
                                              CH)])
        pltpu.sync_copy(oc2, out_hbm.at[pl.ds(plane + 2 * PIX_PER_SAMPLE
                                              + base, CH)])
        return 0

    lax.fori_loop(0, NCHUNK, chunk_body, 0)


@jax.jit
def kernel(img_or_tex, iuv_img, lut):
    tex = img_or_tex.reshape(BATCH * 3 * PIX_PER_SAMPLE)
    lut_a = lut[..., 0].reshape(24 * 256 * 256)
    lut_b = lut[..., 1].reshape(24 * 256 * 256)
    iuv = iuv_img.reshape(BATCH * 3 * PIX_PER_SAMPLE)

    mesh = plsc.VectorSubcoreMesh(
        core_axis_name="c", subcore_axis_name="s",
        num_cores=NC, num_subcores=NS)
    run = pl.kernel(
        _body,
        out_type=jax.ShapeDtypeStruct((BATCH * 3 * PIX_PER_SAMPLE,),
                                      jnp.float32),
        mesh=mesh,
        compiler_params=pltpu.CompilerParams(
            needs_layout_passes=False, use_tc_tiling_on_sc=False),
        scratch_types=[
            pltpu.VMEM((CH,), jnp.int32),
            pltpu.VMEM((CH,), jnp.int32),
            pltpu.VMEM((CH,), jnp.int32),
            pltpu.VMEM((CH,), jnp.int32),
            pltpu.VMEM((CH,), jnp.float32),
            pltpu.VMEM((CH,), jnp.float32),
            pltpu.VMEM((CH,), jnp.int32),
            pltpu.VMEM((CH,), jnp.int32),
            pltpu.VMEM((CH,), jnp.int32),
            pltpu.VMEM((CH,), jnp.float32),
            pltpu.VMEM((CH,), jnp.float32),
            pltpu.VMEM((CH,), jnp.float32),
            pltpu.VMEM((CH,), jnp.float32),
            pltpu.SemaphoreType.DMA,
        ],
    )
    out = run(tex, lut_a, lut_b, iuv)
    return out.reshape(BATCH, 3, H, W)


# speedup vs baseline: 8.5323x; 1.0056x over previous
import jax
import jax.numpy as jnp
import numpy as np
from jax import lax
from jax.experimental import pallas as pl
from jax.experimental.pallas import tpu as pltpu
from jax.experimental.pallas import tpu_sc as plsc

NC = 2
NS = 16
L = 16
NW = NC * NS

H = W = 512
PIX_PER_SAMPLE = H * W
BATCH = 8
TOTAL_PIX = BATCH * PIX_PER_SAMPLE
PIX_PER_WORKER = TOTAL_PIX // NW
CH = 4096
NCHUNK = PIX_PER_WORKER // CH
NIDX = 512
NSTREAM = CH // NIDX
NVREG = CH // L

MAGIC = np.float32(12582912.0)


def _rne(x):
    return (x + MAGIC) - MAGIC


def _body(tex_hbm, luta_hbm, lutb_hbm, iuv_hbm, out_hbm,
          iuv0, iuv1, iuv2, la, arows, brows, t0, t1, t2, mf,
          oc0, oc1, oc2, sem):
    wid = lax.axis_index("s") * NC + lax.axis_index("c")
    n = wid // 4
    q = wid % 4
    n_base_f = (n * 3 * PIX_PER_SAMPLE).astype(jnp.float32)
    plane = n * (3 * PIX_PER_SAMPLE)

    def fire(tbl, idx_ref, dst_ref, j):
        pltpu.async_copy(tbl.at[idx_ref.at[pl.ds(j * NIDX, NIDX)]],
                         dst_ref.at[pl.ds(j * NIDX, NIDX)], sem)

    def drain(tbl, idx_ref, dst_ref, j):
        pltpu.make_async_copy(
            tbl.at[idx_ref.at[pl.ds(j * NIDX, NIDX)]],
            dst_ref.at[pl.ds(j * NIDX, NIDX)], sem).wait()

    def chunk_body(c, _):
        base = q * PIX_PER_WORKER + c * CH

        pltpu.sync_copy(iuv_hbm.at[pl.ds(plane + base, CH)], iuv0)
        pltpu.sync_copy(iuv_hbm.at[pl.ds(plane + PIX_PER_SAMPLE + base, CH)],
                        iuv1)
        pltpu.sync_copy(iuv_hbm.at[pl.ds(plane + 2 * PIX_PER_SAMPLE + base,
                                         CH)], iuv2)

        def pass_a(i, _):
            s = pl.ds(i * L, L)
            p0 = iuv0[s]
            p1 = iuv1[s]
            p2 = iuv2[s]
            part = jnp.where(p0 > 0, p0 - 1, 0)
            la[s] = (part << 16) | (p2 << 8) | p1
            return 0
        lax.fori_loop(0, NVREG, pass_a, 0)

        def fire_lut(j, _):
            fire(luta_hbm, la, arows, j)
            fire(lutb_hbm, la, brows, j)
            return 0
        lax.fori_loop(0, NSTREAM, fire_lut, 0)

        def drain_lut(j, _):
            drain(luta_hbm, la, arows, j)
            drain(lutb_hbm, la, brows, j)
            return 0
        lax.fori_loop(0, NSTREAM, drain_lut, 0)

        def pass_b(i, _):
            s = pl.ds(i * L, L)
            a = arows[s]
            b = brows[s]
            u_i = _rne(a * np.float32(511.0))
            v_i = _rne((np.float32(1.0) - b) * np.float32(511.0))
            tf = v_i * np.float32(512.0) + u_i + n_base_f
            ti = tf.astype(jnp.int32)
            t0[s] = ti
            t1[s] = ti + PIX_PER_SAMPLE
            t2[s] = ti + 2 * PIX_PER_SAMPLE
            p0 = iuv0[s]
            mf[s] = jnp.where(p0 > 0, np.float32(1.0), np.float32(0.0))
            return 0
        lax.fori_loop(0, NVREG, pass_b, 0)

        def fire_tex(j, _):
            fire(tex_hbm, t0, oc0, j)
            fire(tex_hbm, t1, oc1, j)
            fire(tex_hbm, t2, oc2, j)
            return 0
        lax.fori_loop(0, NSTREAM, fire_tex, 0)

        def drain_tex(j, _):
            drain(tex_hbm, t0, oc0, j)
            drain(tex_hbm, t1, oc1, j)
            drain(tex_hbm, t2, oc2, j)
            return 0
        lax.fori_loop(0, NSTREAM, drain_tex, 0)

        def pass_c(i, _):
            s = pl.ds(i * L, L)
            m = mf[s]
            oc0[s] = oc0[s] * m
            oc1[s] = oc1[s] * m
            oc2[s] = oc2[s] * m
            return 0
        lax.fori_loop(0, NVREG, pass_c, 0)

        pltpu.sync_copy(oc0, out_hbm.at[pl.ds(plane + base, CH)])
        pltpu.sync_copy(oc1, out_hbm.at[pl.ds(plane + PIX_PER_SAMPLE + base,
                                              CH)])
        pltpu.sync_copy(oc2, out_hbm.at[pl.ds(plane + 2 * PIX_PER_SAMPLE
                                              + base, CH)])
        return 0

    lax.fori_loop(0, NCHUNK, chunk_body, 0)


@jax.jit
def kernel(img_or_tex, iuv_img, lut):
    tex = img_or_tex.reshape(BATCH * 3 * PIX_PER_SAMPLE)
    lut_a = lut[..., 0].reshape(24 * 256 * 256)
    lut_b = lut[..., 1].reshape(24 * 256 * 256)
    iuv = iuv_img.reshape(BATCH * 3 * PIX_PER_SAMPLE)

    mesh = plsc.VectorSubcoreMesh(
        core_axis_name="c", subcore_axis_name="s",
        num_cores=NC, num_subcores=NS)
    run = pl.kernel(
        _body,
        out_type=jax.ShapeDtypeStruct((BATCH * 3 * PIX_PER_SAMPLE,),
                                      jnp.float32),
        mesh=mesh,
        compiler_params=pltpu.CompilerParams(
            needs_layout_passes=False, use_tc_tiling_on_sc=False,
            disable_bounds_checks=True),
        scratch_types=[
            pltpu.VMEM((CH,), jnp.int32),
            pltpu.VMEM((CH,), jnp.int32),
            pltpu.VMEM((CH,), jnp.int32),
            pltpu.VMEM((CH,), jnp.int32),
            pltpu.VMEM((CH,), jnp.float32),
            pltpu.VMEM((CH,), jnp.float32),
            pltpu.VMEM((CH,), jnp.int32),
            pltpu.VMEM((CH,), jnp.int32),
            pltpu.VMEM((CH,), jnp.int32),
            pltpu.VMEM((CH,), jnp.float32),
            pltpu.VMEM((CH,), jnp.float32),
            pltpu.VMEM((CH,), jnp.float32),
            pltpu.VMEM((CH,), jnp.float32),
            pltpu.SemaphoreType.DMA,
        ],
    )
    out = run(tex, lut_a, lut_b, iuv)
    return out.reshape(BATCH, 3, H, W)


# speedup vs baseline: 9.1196x; 1.0688x over previous
import jax
import jax.numpy as jnp
import numpy as np
from jax import lax
from jax.experimental import pallas as pl
from jax.experimental.pallas import tpu as pltpu
from jax.experimental.pallas import tpu_sc as plsc

NC = 2
NS = 16
L = 16
NW = NC * NS

H = W = 512
PIX_PER_SAMPLE = H * W
BATCH = 8
TOTAL_PIX = BATCH * PIX_PER_SAMPLE
PIX_PER_WORKER = TOTAL_PIX // NW
CH = 4096
NCHUNK = PIX_PER_WORKER // CH
NPAIR = NCHUNK // 2
NIDX = 512
NSTREAM = CH // NIDX
NVREG = CH // L

MAGIC = np.float32(12582912.0)


def _rne(x):
    return (x + MAGIC) - MAGIC


def _body(tex_hbm, luta_hbm, lutb_hbm, iuv_hbm, out_hbm,
          iuv0, iuv1, iuv2,
          la_e, arows_e, brows_e, t0_e, mf_e, oc0_e, oc1_e, oc2_e,
          la_o, arows_o, brows_o, t0_o, mf_o, oc0_o, oc1_o, oc2_o,
          sem_lut_e, sem_lut_o, sem_tex_e, sem_tex_o):
    wid = lax.axis_index("s") * NC + lax.axis_index("c")
    n = wid // 4
    q = wid % 4
    n_base_f = (n * 3 * PIX_PER_SAMPLE).astype(jnp.float32)
    plane = n * (3 * PIX_PER_SAMPLE)

    tex_c1 = tex_hbm.at[pl.ds(PIX_PER_SAMPLE, 23 * PIX_PER_SAMPLE)]
    tex_c2 = tex_hbm.at[pl.ds(2 * PIX_PER_SAMPLE, 22 * PIX_PER_SAMPLE)]

    def fire(tbl, idx_ref, dst_ref, sem):
        def go(j, _):
            pltpu.async_copy(tbl.at[idx_ref.at[pl.ds(j * NIDX, NIDX)]],
                             dst_ref.at[pl.ds(j * NIDX, NIDX)], sem)
            return 0
        lax.fori_loop(0, NSTREAM, go, 0)

    def drain(tbl, idx_ref, dst_ref, sem):
        def go(j, _):
            pltpu.make_async_copy(
                tbl.at[idx_ref.at[pl.ds(j * NIDX, NIDX)]],
                dst_ref.at[pl.ds(j * NIDX, NIDX)], sem).wait()
            return 0
        lax.fori_loop(0, NSTREAM, go, 0)

    def load_and_pass_a(c, la, mf):
        base = q * PIX_PER_WORKER + c * CH
        pltpu.sync_copy(iuv_hbm.at[pl.ds(plane + base, CH)], iuv0)
        pltpu.sync_copy(iuv_hbm.at[pl.ds(plane + PIX_PER_SAMPLE + base, CH)],
                        iuv1)
        pltpu.sync_copy(iuv_hbm.at[pl.ds(plane + 2 * PIX_PER_SAMPLE + base,
                                         CH)], iuv2)

        def pass_a(i, _):
            s = pl.ds(i * L, L)
            p0 = iuv0[s]
            p1 = iuv1[s]
            p2 = iuv2[s]
            part = jnp.where(p0 > 0, p0 - 1, 0)
            la[s] = (part << 16) | (p2 << 8) | p1
            mf[s] = jnp.where(p0 > 0, np.float32(1.0), np.float32(0.0))
            return 0
        lax.fori_loop(0, NVREG, pass_a, 0)

    def pass_b(arows, brows, t0):
        def go(i, _):
            s = pl.ds(i * L, L)
            a = arows[s]
            b = brows[s]
            u_i = _rne(a * np.float32(511.0))
            v_i = _rne((np.float32(1.0) - b) * np.float32(511.0))
            tf = v_i * np.float32(512.0) + u_i + n_base_f
            t0[s] = tf.astype(jnp.int32)
            return 0
        lax.fori_loop(0, NVREG, go, 0)

    def pass_c_and_out(c, mf, oc0, oc1, oc2):
        def go(i, _):
            s = pl.ds(i * L, L)
            m = mf[s]
            oc0[s] = oc0[s] * m
            oc1[s] = oc1[s] * m
            oc2[s] = oc2[s] * m
            return 0
        lax.fori_loop(0, NVREG, go, 0)
        base = q * PIX_PER_WORKER + c * CH
        pltpu.sync_copy(oc0, out_hbm.at[pl.ds(plane + base, CH)])
        pltpu.sync_copy(oc1, out_hbm.at[pl.ds(plane + PIX_PER_SAMPLE + base,
                                              CH)])
        pltpu.sync_copy(oc2, out_hbm.at[pl.ds(plane + 2 * PIX_PER_SAMPLE
                                              + base, CH)])

    def pair_body(k, _):
        ce = 2 * k
        co = 2 * k + 1

        load_and_pass_a(ce, la_e, mf_e)
        fire(luta_hbm, la_e, arows_e, sem_lut_e)
        fire(lutb_hbm, la_e, brows_e, sem_lut_e)

        load_and_pass_a(co, la_o, mf_o)
        fire(luta_hbm, la_o, arows_o, sem_lut_o)
        fire(lutb_hbm, la_o, brows_o, sem_lut_o)

        drain(luta_hbm, la_e, arows_e, sem_lut_e)
        drain(lutb_hbm, la_e, brows_e, sem_lut_e)
        pass_b(arows_e, brows_e, t0_e)
        fire(tex_hbm, t0_e, oc0_e, sem_tex_e)
        fire(tex_c1, t0_e, oc1_e, sem_tex_e)
        fire(tex_c2, t0_e, oc2_e, sem_tex_e)

        drain(luta_hbm, la_o, arows_o, sem_lut_o)
        drain(lutb_hbm, la_o, brows_o, sem_lut_o)
        pass_b(arows_o, brows_o, t0_o)
        fire(tex_hbm, t0_o, oc0_o, sem_tex_o)
        fire(tex_c1, t0_o, oc1_o, sem_tex_o)
        fire(tex_c2, t0_o, oc2_o, sem_tex_o)

        drain(tex_hbm, t0_e, oc0_e, sem_tex_e)
        drain(tex_c1, t0_e, oc1_e, sem_tex_e)
        drain(tex_c2, t0_e, oc2_e, sem_tex_e)
        pass_c_and_out(ce, mf_e, oc0_e, oc1_e, oc2_e)

        drain(tex_hbm, t0_o, oc0_o, sem_tex_o)
        drain(tex_c1, t0_o, oc1_o, sem_tex_o)
        drain(tex_c2, t0_o, oc2_o, sem_tex_o)
        pass_c_and_out(co, mf_o, oc0_o, oc1_o, oc2_o)
        return 0

    lax.fori_loop(0, NPAIR, pair_body, 0)


@jax.jit
def kernel(img_or_tex, iuv_img, lut):
    tex = img_or_tex.reshape(BATCH * 3 * PIX_PER_SAMPLE)
    lut_a = lut[..., 0].reshape(24 * 256 * 256)
    lut_b = lut[..., 1].reshape(24 * 256 * 256)
    iuv = iuv_img.reshape(BATCH * 3 * PIX_PER_SAMPLE)

    mesh = plsc.VectorSubcoreMesh(
        core_axis_name="c", subcore_axis_name="s",
        num_cores=NC, num_subcores=NS)
    dbl = [
        pltpu.VMEM((CH,), jnp.int32),
        pltpu.VMEM((CH,), jnp.float32),
        pltpu.VMEM((CH,), jnp.float32),
        pltpu.VMEM((CH,), jnp.int32),
        pltpu.VMEM((CH,), jnp.float32),
        pltpu.VMEM((CH,), jnp.float32),
        pltpu.VMEM((CH,), jnp.float32),
        pltpu.VMEM((CH,), jnp.float32),
    ]
    run = pl.kernel(
        _body,
        out_type=jax.ShapeDtypeStruct((BATCH * 3 * PIX_PER_SAMPLE,),
                                      jnp.float32),
        mesh=mesh,
        compiler_params=pltpu.CompilerParams(
            needs_layout_passes=False, use_tc_tiling_on_sc=False,
            disable_bounds_checks=True),
        scratch_types=(
            [pltpu.VMEM((CH,), jnp.int32)] * 3
            + dbl + dbl
            + [pltpu.SemaphoreType.DMA] * 4
        ),
    )
    out = run(tex, lut_a, lut_b, iuv)
    return out.reshape(BATCH, 3, H, W)


# speedup vs baseline: 9.1535x; 1.0037x over previous
import jax
import jax.numpy as jnp
import numpy as np
from jax import lax
from jax.experimental import pallas as pl
from jax.experimental.pallas import tpu as pltpu
from jax.experimental.pallas import tpu_sc as plsc

NC = 2
NS = 16
L = 16
NW = NC * NS

H = W = 512
PIX_PER_SAMPLE = H * W
BATCH = 8
TOTAL_PIX = BATCH * PIX_PER_SAMPLE
PIX_PER_WORKER = TOTAL_PIX // NW
CH = 4096
NCHUNK = PIX_PER_WORKER // CH
NPAIR = NCHUNK // 2
NIDX = 512
NSTREAM = CH // NIDX
NVREG = CH // L

MAGIC = np.float32(12582912.0)


def _rne(x):
    return (x + MAGIC) - MAGIC


def _body(tex_hbm, luta_hbm, lutb_hbm, iuv_hbm, out_hbm,
          iuv0, iuv1, iuv2,
          la_e, arows_e, brows_e, t0_e, mf_e, oc0_e, oc1_e, oc2_e,
          la_o, arows_o, brows_o, t0_o, mf_o, oc0_o, oc1_o, oc2_o,
          sem_lut_e, sem_lut_o, sem_tex_e, sem_tex_o):
    wid = lax.axis_index("s") * NC + lax.axis_index("c")
    n = wid // 4
    q = wid % 4
    n_base_f = (n * 3 * PIX_PER_SAMPLE).astype(jnp.float32)
    plane = n * (3 * PIX_PER_SAMPLE)

    tex_c1 = tex_hbm.at[pl.ds(PIX_PER_SAMPLE, 23 * PIX_PER_SAMPLE)]
    tex_c2 = tex_hbm.at[pl.ds(2 * PIX_PER_SAMPLE, 22 * PIX_PER_SAMPLE)]

    def fire(tbl, idx_ref, dst_ref, sem):
        def go(j, _):
            pltpu.async_copy(tbl.at[idx_ref.at[pl.ds(j * NIDX, NIDX)]],
                             dst_ref.at[pl.ds(j * NIDX, NIDX)], sem)
            return 0
        lax.fori_loop(0, NSTREAM, go, 0)

    def drain(tbl, idx_ref, dst_ref, sem):
        def go(j, _):
            pltpu.make_async_copy(
                tbl.at[idx_ref.at[pl.ds(j * NIDX, NIDX)]],
                dst_ref.at[pl.ds(j * NIDX, NIDX)], sem).wait()
            return 0
        lax.fori_loop(0, NSTREAM, go, 0)

    def load_and_pass_a(c, la, mf):
        base = q * PIX_PER_WORKER + c * CH
        pltpu.sync_copy(iuv_hbm.at[pl.ds(plane + base, CH)], iuv0)
        pltpu.sync_copy(iuv_hbm.at[pl.ds(plane + PIX_PER_SAMPLE + base, CH)],
                        iuv1)
        pltpu.sync_copy(iuv_hbm.at[pl.ds(plane + 2 * PIX_PER_SAMPLE + base,
                                         CH)], iuv2)

        @plsc.parallel_loop(0, CH, L, unroll=4)
        def pass_a(o):
            s = pl.ds(o, L)
            p0 = iuv0[s]
            p1 = iuv1[s]
            p2 = iuv2[s]
            part = jnp.where(p0 > 0, p0 - 1, 0)
            la[s] = (part << 16) | (p2 << 8) | p1
            mf[s] = jnp.where(p0 > 0, np.float32(1.0), np.float32(0.0))

    def pass_b(arows, brows, t0):
        @plsc.parallel_loop(0, CH, L, unroll=4)
        def go(o):
            s = pl.ds(o, L)
            a = arows[s]
            b = brows[s]
            u_i = _rne(a * np.float32(511.0))
            v_i = _rne((np.float32(1.0) - b) * np.float32(511.0))
            tf = v_i * np.float32(512.0) + u_i + n_base_f
            t0[s] = tf.astype(jnp.int32)

    def pass_c_and_out(c, mf, oc0, oc1, oc2):
        @plsc.parallel_loop(0, CH, L, unroll=4)
        def go(o):
            s = pl.ds(o, L)
            m = mf[s]
            oc0[s] = oc0[s] * m
            oc1[s] = oc1[s] * m
            oc2[s] = oc2[s] * m
        base = q * PIX_PER_WORKER + c * CH
        pltpu.sync_copy(oc0, out_hbm.at[pl.ds(plane + base, CH)])
        pltpu.sync_copy(oc1, out_hbm.at[pl.ds(plane + PIX_PER_SAMPLE + base,
                                              CH)])
        pltpu.sync_copy(oc2, out_hbm.at[pl.ds(plane + 2 * PIX_PER_SAMPLE
                                              + base, CH)])

    def pair_body(k, _):
        ce = 2 * k
        co = 2 * k + 1

        load_and_pass_a(ce, la_e, mf_e)
        fire(luta_hbm, la_e, arows_e, sem_lut_e)
        fire(lutb_hbm, la_e, brows_e, sem_lut_e)

        load_and_pass_a(co, la_o, mf_o)
        fire(luta_hbm, la_o, arows_o, sem_lut_o)
        fire(lutb_hbm, la_o, brows_o, sem_lut_o)

        drain(luta_hbm, la_e, arows_e, sem_lut_e)
        drain(lutb_hbm, la_e, brows_e, sem_lut_e)
        pass_b(arows_e, brows_e, t0_e)
        fire(tex_hbm, t0_e, oc0_e, sem_tex_e)
        fire(tex_c1, t0_e, oc1_e, sem_tex_e)
        fire(tex_c2, t0_e, oc2_e, sem_tex_e)

        drain(luta_hbm, la_o, arows_o, sem_lut_o)
        drain(lutb_hbm, la_o, brows_o, sem_lut_o)
        pass_b(arows_o, brows_o, t0_o)
        fire(tex_hbm, t0_o, oc0_o, sem_tex_o)
        fire(tex_c1, t0_o, oc1_o, sem_tex_o)
        fire(tex_c2, t0_o, oc2_o, sem_tex_o)

        drain(tex_hbm, t0_e, oc0_e, sem_tex_e)
        drain(tex_c1, t0_e, oc1_e, sem_tex_e)
        drain(tex_c2, t0_e, oc2_e, sem_tex_e)
        pass_c_and_out(ce, mf_e, oc0_e, oc1_e, oc2_e)

        drain(tex_hbm, t0_o, oc0_o, sem_tex_o)
        drain(tex_c1, t0_o, oc1_o, sem_tex_o)
        drain(tex_c2, t0_o, oc2_o, sem_tex_o)
        pass_c_and_out(co, mf_o, oc0_o, oc1_o, oc2_o)
        return 0

    lax.fori_loop(0, NPAIR, pair_body, 0)


@jax.jit
def kernel(img_or_tex, iuv_img, lut):
    tex = img_or_tex.reshape(BATCH * 3 * PIX_PER_SAMPLE)
    lut_a = lut[..., 0].reshape(24 * 256 * 256)
    lut_b = lut[..., 1].reshape(24 * 256 * 256)
    iuv = iuv_img.reshape(BATCH * 3 * PIX_PER_SAMPLE)

    mesh = plsc.VectorSubcoreMesh(
        core_axis_name="c", subcore_axis_name="s",
        num_cores=NC, num_subcores=NS)
    dbl = [
        pltpu.VMEM((CH,), jnp.int32),
        pltpu.VMEM((CH,), jnp.float32),
        pltpu.VMEM((CH,), jnp.float32),
        pltpu.VMEM((CH,), jnp.int32),
        pltpu.VMEM((CH,), jnp.float32),
        pltpu.VMEM((CH,), jnp.float32),
        pltpu.VMEM((CH,), jnp.float32),
        pltpu.VMEM((CH,), jnp.float32),
    ]
    run = pl.kernel(
        _body,
        out_type=jax.ShapeDtypeStruct((BATCH * 3 * PIX_PER_SAMPLE,),
                                      jnp.float32),
        mesh=mesh,
        compiler_params=pltpu.CompilerParams(
            needs_layout_passes=False, use_tc_tiling_on_sc=False,
            disable_bounds_checks=True),
        scratch_types=(
            [pltpu.VMEM((CH,), jnp.int32)] * 3
            + dbl + dbl
            + [pltpu.SemaphoreType.DMA] * 4
        ),
    )
    out = run(tex, lut_a, lut_b, iuv)
    return out.reshape(BATCH, 3, H, W)


# speedup vs baseline: 9.1687x; 1.0017x over previous
import jax
import jax.numpy as jnp
import numpy as np
from jax import lax
from jax.experimental import pallas as pl
from jax.experimental.pallas import tpu as pltpu
from jax.experimental.pallas import tpu_sc as plsc

NC = 2
NS = 16
L = 16
NW = NC * NS

H = W = 512
PIX_PER_SAMPLE = H * W
BATCH = 8
TOTAL_PIX = BATCH * PIX_PER_SAMPLE
PIX_PER_WORKER = TOTAL_PIX // NW
CH = 4096
NCHUNK = PIX_PER_WORKER // CH
NPAIR = NCHUNK // 2
NIDX = 512
NSTREAM = CH // NIDX
NVREG = CH // L

MAGIC = np.float32(12582912.0)


def _rne(x):
    return (x + MAGIC) - MAGIC


def _body(tex_hbm, luta_hbm, lutb_hbm, iuv_hbm, out_hbm,
          iuv0, iuv1, iuv2,
          la_e, arows_e, brows_e, t0_e, mf_e, oc0_e, oc1_e, oc2_e,
          la_o, arows_o, brows_o, t0_o, mf_o, oc0_o, oc1_o, oc2_o,
          sem_lut_e, sem_lut_o, sem_tex_e, sem_tex_o):
    wid = lax.axis_index("s") * NC + lax.axis_index("c")
    n = wid // 4
    q = wid % 4
    n_base_f = (n * 3 * PIX_PER_SAMPLE).astype(jnp.float32)
    plane = n * (3 * PIX_PER_SAMPLE)

    tex_c1 = tex_hbm.at[pl.ds(PIX_PER_SAMPLE, 23 * PIX_PER_SAMPLE)]
    tex_c2 = tex_hbm.at[pl.ds(2 * PIX_PER_SAMPLE, 22 * PIX_PER_SAMPLE)]

    def fire(tbl, idx_ref, dst_ref, sem):
        def go(j, _):
            pltpu.async_copy(tbl.at[idx_ref.at[pl.ds(j * NIDX, NIDX)]],
                             dst_ref.at[pl.ds(j * NIDX, NIDX)], sem)
            return 0
        lax.fori_loop(0, NSTREAM, go, 0)

    def drain(tbl, idx_ref, dst_ref, sem):
        def go(j, _):
            pltpu.make_async_copy(
                tbl.at[idx_ref.at[pl.ds(j * NIDX, NIDX)]],
                dst_ref.at[pl.ds(j * NIDX, NIDX)], sem).wait()
            return 0
        lax.fori_loop(0, NSTREAM, go, 0)

    def load_and_pass_a(c, la, mf):
        base = q * PIX_PER_WORKER + c * CH
        pltpu.sync_copy(iuv_hbm.at[pl.ds(plane + base, CH)], iuv0)
        pltpu.sync_copy(iuv_hbm.at[pl.ds(plane + PIX_PER_SAMPLE + base, CH)],
                        iuv1)
        pltpu.sync_copy(iuv_hbm.at[pl.ds(plane + 2 * PIX_PER_SAMPLE + base,
                                         CH)], iuv2)

        def pass_a(i, _):
            s = pl.ds(i * L, L)
            p0 = iuv0[s]
            p1 = iuv1[s]
            p2 = iuv2[s]
            part = jnp.where(p0 > 0, p0 - 1, 0)
            la[s] = (part << 16) | (p2 << 8) | p1
            mf[s] = jnp.where(p0 > 0, np.float32(1.0), np.float32(0.0))
            return 0
        lax.fori_loop(0, NVREG, pass_a, 0)

    def pass_b(arows, brows, t0):
        def go(i, _):
            s = pl.ds(i * L, L)
            a = arows[s]
            b = brows[s]
            u_i = _rne(a * np.float32(511.0))
            v_i = _rne((np.float32(1.0) - b) * np.float32(511.0))
            tf = v_i * np.float32(512.0) + u_i + n_base_f
            t0[s] = tf.astype(jnp.int32)
            return 0
        lax.fori_loop(0, NVREG, go, 0)

    def pass_c_and_out(c, mf, oc0, oc1, oc2):
        def go(i, _):
            s = pl.ds(i * L, L)
            m = mf[s]
            oc0[s] = oc0[s] * m
            oc1[s] = oc1[s] * m
            oc2[s] = oc2[s] * m
            return 0
        lax.fori_loop(0, NVREG, go, 0)
        base = q * PIX_PER_WORKER + c * CH
        pltpu.sync_copy(oc0, out_hbm.at[pl.ds(plane + base, CH)])
        pltpu.sync_copy(oc1, out_hbm.at[pl.ds(plane + PIX_PER_SAMPLE + base,
                                              CH)])
        pltpu.sync_copy(oc2, out_hbm.at[pl.ds(plane + 2 * PIX_PER_SAMPLE
                                              + base, CH)])

    load_and_pass_a(0, la_e, mf_e)
    fire(luta_hbm, la_e, arows_e, sem_lut_e)
    fire(lutb_hbm, la_e, brows_e, sem_lut_e)
    load_and_pass_a(1, la_o, mf_o)
    fire(luta_hbm, la_o, arows_o, sem_lut_o)
    fire(lutb_hbm, la_o, brows_o, sem_lut_o)

    def pair_body(k, _):
        ce = 2 * k
        co = 2 * k + 1

        drain(luta_hbm, la_e, arows_e, sem_lut_e)
        drain(lutb_hbm, la_e, brows_e, sem_lut_e)
        pass_b(arows_e, brows_e, t0_e)
        fire(tex_hbm, t0_e, oc0_e, sem_tex_e)
        fire(tex_c1, t0_e, oc1_e, sem_tex_e)
        fire(tex_c2, t0_e, oc2_e, sem_tex_e)

        drain(luta_hbm, la_o, arows_o, sem_lut_o)
        drain(lutb_hbm, la_o, brows_o, sem_lut_o)
        pass_b(arows_o, brows_o, t0_o)
        fire(tex_hbm, t0_o, oc0_o, sem_tex_o)
        fire(tex_c1, t0_o, oc1_o, sem_tex_o)
        fire(tex_c2, t0_o, oc2_o, sem_tex_o)

        drain(tex_hbm, t0_e, oc0_e, sem_tex_e)
        drain(tex_c1, t0_e, oc1_e, sem_tex_e)
        drain(tex_c2, t0_e, oc2_e, sem_tex_e)
        pass_c_and_out(ce, mf_e, oc0_e, oc1_e, oc2_e)

        @pl.when(k < NPAIR - 1)
        def _():
            load_and_pass_a(ce + 2, la_e, mf_e)
            fire(luta_hbm, la_e, arows_e, sem_lut_e)
            fire(lutb_hbm, la_e, brows_e, sem_lut_e)

        drain(tex_hbm, t0_o, oc0_o, sem_tex_o)
        drain(tex_c1, t0_o, oc1_o, sem_tex_o)
        drain(tex_c2, t0_o, oc2_o, sem_tex_o)
        pass_c_and_out(co, mf_o, oc0_o, oc1_o, oc2_o)

        @pl.when(k < NPAIR - 1)
        def _():
            load_and_pass_a(co + 2, la_o, mf_o)
            fire(luta_hbm, la_o, arows_o, sem_lut_o)
            fire(lutb_hbm, la_o, brows_o, sem_lut_o)
        return 0

    lax.fori_loop(0, NPAIR, pair_body, 0)


@jax.jit
def kernel(img_or_tex, iuv_img, lut):
    tex = img_or_tex.reshape(BATCH * 3 * PIX_PER_SAMPLE)
    lut_a = lut[..., 0].reshape(24 * 256 * 256)
    lut_b = lut[..., 1].reshape(24 * 256 * 256)
    iuv = iuv_img.reshape(BATCH * 3 * PIX_PER_SAMPLE)

    mesh = plsc.VectorSubcoreMesh(
        core_axis_name="c", subcore_axis_name="s",
        num_cores=NC, num_subcores=NS)
    dbl = [
        pltpu.VMEM((CH,), jnp.int32),
        pltpu.VMEM((CH,), jnp.float32),
        pltpu.VMEM((CH,), jnp.float32),
        pltpu.VMEM((CH,), jnp.int32),
        pltpu.VMEM((CH,), jnp.float32),
        pltpu.VMEM((CH,), jnp.float32),
        pltpu.VMEM((CH,), jnp.float32),
        pltpu.VMEM((CH,), jnp.float32),
    ]
    run = pl.kernel(
        _body,
        out_type=jax.ShapeDtypeStruct((BATCH * 3 * PIX_PER_SAMPLE,),
                                      jnp.float32),
        mesh=mesh,
        compiler_params=pltpu.CompilerParams(
            needs_layout_passes=False, use_tc_tiling_on_sc=False,
            disable_bounds_checks=True),
        scratch_types=(
            [pltpu.VMEM((CH,), jnp.int32)] * 3
            + dbl + dbl
            + [pltpu.SemaphoreType.DMA] * 4
        ),
    )
    out = run(tex, lut_a, lut_b, iuv)
    return out.reshape(BATCH, 3, H, W)


# speedup vs baseline: 10.4084x; 1.1352x over previous
import jax
import jax.numpy as jnp
import numpy as np
from jax import lax
from jax.experimental import pallas as pl
from jax.experimental.pallas import tpu as pltpu
from jax.experimental.pallas import tpu_sc as plsc

NC = 2
NS = 16
L = 16
NW = NC * NS

H = W = 512
PIX_PER_SAMPLE = H * W
BATCH = 8
TOTAL_PIX = BATCH * PIX_PER_SAMPLE
PIX_PER_WORKER = TOTAL_PIX // NW
CH = 4096
NCHUNK = PIX_PER_WORKER // CH
NPAIR = NCHUNK // 2
NIDX = 512
NSTREAM = CH // NIDX
NVREG = CH // L

MAGIC = np.float32(12582912.0)


def _rne(x):
    return (x + MAGIC) - MAGIC


def _body(tex_hbm, luta_hbm, lutb_hbm, iuv_hbm, out_hbm,
          iuv0, iuv1, iuv2,
          la_e, arows_e, brows_e, t0_e, mf_e, oc0_e, oc1_e, oc2_e,
          la_o, arows_o, brows_o, t0_o, mf_o, oc0_o, oc1_o, oc2_o,
          sem_lut_e, sem_lut_o, sem_tex_e, sem_tex_o):
    wid = lax.axis_index("s") * NC + lax.axis_index("c")
    n = wid // 4
    q = wid % 4
    n_base_f = (n * 3 * PIX_PER_SAMPLE).astype(jnp.float32)
    plane = n * (3 * PIX_PER_SAMPLE)

    tex_c1 = tex_hbm.at[pl.ds(PIX_PER_SAMPLE, 23 * PIX_PER_SAMPLE)]
    tex_c2 = tex_hbm.at[pl.ds(2 * PIX_PER_SAMPLE, 22 * PIX_PER_SAMPLE)]

    def fire(tbl, idx_ref, dst_ref, sem):
        def go(j, _):
            pltpu.async_copy(tbl.at[idx_ref.at[pl.ds(j * NIDX, NIDX)]],
                             dst_ref.at[pl.ds(j * NIDX, NIDX)], sem)
            return 0
        lax.fori_loop(0, NSTREAM, go, 0)

    def drain(tbl, idx_ref, dst_ref, sem):
        def go(j, _):
            pltpu.make_async_copy(
                tbl.at[idx_ref.at[pl.ds(j * NIDX, NIDX)]],
                dst_ref.at[pl.ds(j * NIDX, NIDX)], sem).wait()
            return 0
        lax.fori_loop(0, NSTREAM, go, 0)

    def load_and_pass_a(c, la, mf):
        base = q * PIX_PER_WORKER + c * CH
        pltpu.sync_copy(iuv_hbm.at[pl.ds(plane + base, CH)], iuv0)
        pltpu.sync_copy(iuv_hbm.at[pl.ds(plane + PIX_PER_SAMPLE + base, CH)],
                        iuv1)
        pltpu.sync_copy(iuv_hbm.at[pl.ds(plane + 2 * PIX_PER_SAMPLE + base,
                                         CH)], iuv2)

        def pass_a(i, _):
            s = pl.ds(i * L, L)
            p0 = iuv0[s]
            p1 = iuv1[s]
            p2 = iuv2[s]
            part = jnp.where(p0 > 0, p0 - 1, 0)
            la[s] = (part << 16) | (p2 << 8) | p1
            mf[s] = jnp.where(p0 > 0, np.float32(1.0), np.float32(0.0))
            return 0
        lax.fori_loop(0, NVREG, pass_a, 0)

    def pass_b(arows, brows, t0):
        def go(i, _):
            s = pl.ds(i * L, L)
            a = arows[s]
            b = brows[s]
            u_i = _rne(a * np.float32(511.0))
            v_i = _rne((np.float32(1.0) - b) * np.float32(511.0))
            tf = v_i * np.float32(512.0) + u_i + n_base_f
            t0[s] = tf.astype(jnp.int32)
            return 0
        lax.fori_loop(0, NVREG, go, 0)

    def pass_c_and_out(c, mf, oc0, oc1, oc2):
        def go(i, _):
            s = pl.ds(i * L, L)
            m = mf[s]
            oc0[s] = oc0[s] * m
            oc1[s] = oc1[s] * m
            oc2[s] = oc2[s] * m
            return 0
        lax.fori_loop(0, NVREG, go, 0)
        base = q * PIX_PER_WORKER + c * CH
        pltpu.sync_copy(oc0, out_hbm.at[pl.ds(plane + base, CH)])
        pltpu.sync_copy(oc1, out_hbm.at[pl.ds(plane + PIX_PER_SAMPLE + base,
                                              CH)])
        pltpu.sync_copy(oc2, out_hbm.at[pl.ds(plane + 2 * PIX_PER_SAMPLE
                                              + base, CH)])

    load_and_pass_a(0, la_e, mf_e)
    fire(luta_hbm, la_e, arows_e, sem_lut_e)
    fire(lutb_hbm, la_e, brows_e, sem_lut_e)
    load_and_pass_a(1, la_o, mf_o)
    fire(luta_hbm, la_o, arows_o, sem_lut_o)
    fire(lutb_hbm, la_o, brows_o, sem_lut_o)

    def pair_body(k, _):
        ce = 2 * k
        co = 2 * k + 1

        drain(luta_hbm, la_e, arows_e, sem_lut_e)
        drain(lutb_hbm, la_e, brows_e, sem_lut_e)
        pass_b(arows_e, brows_e, t0_e)
        fire(tex_hbm, t0_e, oc0_e, sem_tex_e)
        fire(tex_c1, t0_e, oc1_e, sem_tex_e)
        fire(tex_c2, t0_e, oc2_e, sem_tex_e)

        drain(luta_hbm, la_o, arows_o, sem_lut_o)
        drain(lutb_hbm, la_o, brows_o, sem_lut_o)
        pass_b(arows_o, brows_o, t0_o)
        fire(tex_hbm, t0_o, oc0_o, sem_tex_o)
        fire(tex_c1, t0_o, oc1_o, sem_tex_o)
        fire(tex_c2, t0_o, oc2_o, sem_tex_o)

        drain(tex_hbm, t0_e, oc0_e, sem_tex_e)
        drain(tex_c1, t0_e, oc1_e, sem_tex_e)
        drain(tex_c2, t0_e, oc2_e, sem_tex_e)
        pass_c_and_out(ce, mf_e, oc0_e, oc1_e, oc2_e)

        @pl.when(k < NPAIR - 1)
        def _():
            load_and_pass_a(ce + 2, la_e, mf_e)
            fire(luta_hbm, la_e, arows_e, sem_lut_e)
            fire(lutb_hbm, la_e, brows_e, sem_lut_e)

        drain(tex_hbm, t0_o, oc0_o, sem_tex_o)
        drain(tex_c1, t0_o, oc1_o, sem_tex_o)
        drain(tex_c2, t0_o, oc2_o, sem_tex_o)
        pass_c_and_out(co, mf_o, oc0_o, oc1_o, oc2_o)

        @pl.when(k < NPAIR - 1)
        def _():
            load_and_pass_a(co + 2, la_o, mf_o)
            fire(luta_hbm, la_o, arows_o, sem_lut_o)
            fire(lutb_hbm, la_o, brows_o, sem_lut_o)
        return 0

    lax.fori_loop(0, NPAIR, pair_body, 0)


@jax.jit
def kernel(img_or_tex, iuv_img, lut):
    tex = img_or_tex.reshape(BATCH * 3 * PIX_PER_SAMPLE)
    lut_a = lut[..., 0].reshape(24 * 256 * 256)
    lut_b = lut[..., 1].reshape(24 * 256 * 256)
    iuv = (iuv_img.reshape(BATCH, 3, 64, 8, 4, 128)
           .transpose(0, 1, 2, 4, 3, 5)
           .reshape(BATCH * 3 * PIX_PER_SAMPLE))

    mesh = plsc.VectorSubcoreMesh(
        core_axis_name="c", subcore_axis_name="s",
        num_cores=NC, num_subcores=NS)
    dbl = [
        pltpu.VMEM((CH,), jnp.int32),
        pltpu.VMEM((CH,), jnp.float32),
        pltpu.VMEM((CH,), jnp.float32),
        pltpu.VMEM((CH,), jnp.int32),
        pltpu.VMEM((CH,), jnp.float32),
        pltpu.VMEM((CH,), jnp.float32),
        pltpu.VMEM((CH,), jnp.float32),
        pltpu.VMEM((CH,), jnp.float32),
    ]
    run = pl.kernel(
        _body,
        out_type=jax.ShapeDtypeStruct((BATCH * 3 * PIX_PER_SAMPLE,),
                                      jnp.float32),
        mesh=mesh,
        compiler_params=pltpu.CompilerParams(
            needs_layout_passes=False, use_tc_tiling_on_sc=False,
            disable_bounds_checks=True),
        scratch_types=(
            [pltpu.VMEM((CH,), jnp.int32)] * 3
            + dbl + dbl
            + [pltpu.SemaphoreType.DMA] * 4
        ),
    )
    out = run(tex, lut_a, lut_b, iuv)
    return (out.reshape(BATCH, 3, 64, 4, 8, 128)
            .transpose(0, 1, 2, 4, 3, 5)
            .reshape(BATCH, 3, H, W))
